# baseline (speedup 1.0000x reference)
import functools

import jax
import jax.numpy as jnp
from jax import lax
from jax.experimental import pallas as pl
from jax.experimental.pallas import tpu as pltpu
from jax.experimental.pallas import tpu_sc as plsc

NUM_FIELDS = 26
FEAT_NUM = 100000
FEATURE_LENGTH = NUM_FIELDS * FEAT_NUM
K = 16
BATCH = 16384

NC = 2
NS = 16
NW = NC * NS
BPW = BATCH // NW
IPW = BPW * NUM_FIELDS
GW = 128
NG = IPW // GW
CHUNK_B = 64
CHUNK_ROWS = CHUNK_B * NUM_FIELDS
CHUNK_G = CHUNK_ROWS // GW
NCHUNK = BPW // CHUNK_B


def _fm_body(inputs_hbm, w0_hbm, w_hbm, v_hbm, out_hbm,
             idx_v, rows_a, rows_b, wval_v, rbuf_v, outbuf_v, w0_v,
             sem_a, sem_b):
    wid = lax.axis_index("s") * NC + lax.axis_index("c")
    iota16 = lax.iota(jnp.int32, 16)

    pltpu.sync_copy(inputs_hbm.at[wid], idx_v)
    pltpu.sync_copy(w0_hbm, w0_v.at[pl.ds(0, 1)])

    def transform(j, carry):
        for l in range(GW // 16):
            sl = idx_v[j, pl.ds(l * 16, 16)]
            p = j * GW + l * 16 + iota16
            fld = lax.rem(p, NUM_FIELDS)
            idx_v[j, pl.ds(l * 16, 16)] = sl + fld * FEAT_NUM
        return carry

    lax.fori_loop(0, NG, transform, 0)

    bufs = (rows_a, rows_b)
    sems = (sem_a, sem_b)

    def fire(c):
        rows = bufs[c % 2]
        sem = sems[c % 2]
        cps = []
        for g in range(CHUNK_G):
            j = c * CHUNK_G + g
            cps.append(pltpu.async_copy(
                v_hbm.at[idx_v.at[j]], rows.at[pl.ds(g * GW, GW)], sem))
            cps.append(pltpu.async_copy(
                w_hbm.at[idx_v.at[j]], wval_v.at[j], sem))
        return cps

    w0s = w0_v[pl.ds(0, 16)][0]

    def compute(c):
        rows = bufs[c % 2]

        def bbody(b, carry):
            base = b * NUM_FIELDS
            r = rows[base, :]
            s = r
            sq = r * r
            for f in range(1, NUM_FIELDS):
                r = rows[base + f, :]
                s = s + r
                sq = sq + r * r
            rbuf_v[b, :] = s * s - sq
            return carry

        lax.fori_loop(0, CHUNK_B, bbody, 0)

        def gbody(g, carry):
            rowv = g * 16 + iota16
            acc = plsc.load_gather(rbuf_v, [rowv, jnp.zeros((16,), jnp.int32)])
            for k in range(1, K):
                acc = acc + plsc.load_gather(
                    rbuf_v, [rowv, jnp.full((16,), k, jnp.int32)])
            q0 = (c * CHUNK_B + g * 16) * NUM_FIELDS + iota16 * NUM_FIELDS
            fo = plsc.load_gather(
                wval_v, [lax.shift_right_logical(q0, 7),
                         lax.bitwise_and(q0, GW - 1)])
            for f in range(1, NUM_FIELDS):
                q = q0 + f
                fo = fo + plsc.load_gather(
                    wval_v, [lax.shift_right_logical(q, 7),
                             lax.bitwise_and(q, GW - 1)])
            outbuf_v[pl.ds(c * CHUNK_B + g * 16, 16)] = w0s + fo + 0.5 * acc
            return carry

        lax.fori_loop(0, CHUNK_B // 16, gbody, 0)

    pending = fire(0)
    for c in range(NCHUNK):
        nxt = fire(c + 1) if c + 1 < NCHUNK else []
        for cp in pending:
            cp.wait()
        pending = nxt
        compute(c)

    pltpu.sync_copy(outbuf_v, out_hbm.at[pl.ds(wid * BPW, BPW)])


@jax.jit
def _fm(inputs_r, w0, w_flat, v):
    mesh = plsc.VectorSubcoreMesh(core_axis_name="c", subcore_axis_name="s")
    f = functools.partial(
        pl.kernel,
        out_type=jax.ShapeDtypeStruct((BATCH,), jnp.float32),
        mesh=mesh,
        compiler_params=pltpu.CompilerParams(
            needs_layout_passes=False, use_tc_tiling_on_sc=False),
        scratch_types=[
            pltpu.VMEM((NG, GW), jnp.int32),
            pltpu.VMEM((CHUNK_ROWS, K), jnp.float32),
            pltpu.VMEM((CHUNK_ROWS, K), jnp.float32),
            pltpu.VMEM((NG, GW), jnp.float32),
            pltpu.VMEM((CHUNK_B, K), jnp.float32),
            pltpu.VMEM((BPW,), jnp.float32),
            pltpu.VMEM((16,), jnp.float32),
            pltpu.SemaphoreType.DMA,
            pltpu.SemaphoreType.DMA,
        ],
    )(_fm_body)
    return f(inputs_r, w0, w_flat, v)


def kernel(inputs, w0, w, V):
    inputs_r = inputs.reshape(NW, NG, GW)
    w_flat = w.reshape(FEATURE_LENGTH)
    out = _fm(inputs_r, w0, w_flat, V)
    return out.reshape(BATCH, 1)
